# baseline (speedup 1.0000x reference)
import jax
import jax.numpy as jnp
from jax.experimental import pallas as pl
from jax.experimental.pallas import tpu as pltpu

_HIDDEN = 128
_NSEG = 1024
_R = 3200
_G1 = _R // 8
_G2 = _G1 // 8
_S = 64
_OFF_REND = _R
_OFF_G1F = 2 * _R
_OFF_C1 = _OFF_G1F + 512
_OFF_G2F = _OFF_C1 + 512
_OFF_C2 = _OFF_G2F + 128
_RP = _OFF_C2 + 128


def _seg_scan_step(vals, ids, d):
    sh_v = jnp.concatenate(
        [jnp.zeros((d, _HIDDEN), jnp.float32), vals[:-d, :]], axis=0)
    sh_i = jnp.concatenate(
        [jnp.full((d, 1), -1, jnp.int32), ids[:-d, :]], axis=0)
    return jnp.where(ids == sh_i, jnp.maximum(vals, sh_v), vals)


def _fused_kernel(wlo_ref, whi_ref, x_ref, bcol_ref, rp_ref, w1_ref, b1_ref,
                  w2_ref, b2_ref, out_ref, sum_s, max_s, cnt_s):
    i = pl.program_id(0)
    nb = pl.num_programs(0)

    @pl.when(i == 0)
    def _init():
        sum_s[...] = jnp.zeros_like(sum_s)
        max_s[...] = jnp.zeros_like(max_s)
        cnt_s[...] = jnp.zeros_like(cnt_s)

    x = x_ref[...]
    h = jax.lax.dot_general(x, w1_ref[...], (((1,), (1,)), ((), ())),
                            preferred_element_type=jnp.float32)
    h = jnp.maximum(h + b1_ref[...], 0.0)
    att = jax.lax.dot_general(h, w2_ref[...], (((1,), (1,)), ((), ())),
                              preferred_element_type=jnp.float32)
    att = jax.nn.sigmoid(att + b2_ref[...])
    attended = h * att

    bcol = bcol_ref[0]
    rp = rp_ref[0]
    brow = rp[:, 0:_R]
    rend = rp[:, _OFF_REND:_OFF_REND + _R]
    g1first = rp[:, _OFF_G1F:_OFF_G1F + _G1]
    cond1 = rp[:, _OFF_C1:_OFF_C1 + _G1]
    g2first = rp[:, _OFF_G2F:_OFF_G2F + _G2]
    cond2 = rp[:, _OFF_C2:_OFF_C2 + _G2]

    s = attended
    for d in (1, 2, 4):
        s = _seg_scan_step(s, bcol, d)
    scanned1 = s

    gsum = jnp.reshape(scanned1, (_G1, 8, _HIDDEN))[:, 7, :]
    g1last = jnp.reshape(bcol, (_G1, 8, 1))[:, 7, :]
    t = gsum
    for d in (1, 2, 4):
        t = _seg_scan_step(t, g1last, d)
    l2prev = jnp.concatenate(
        [jnp.zeros((1, _HIDDEN), jnp.float32), t[:-1, :]], axis=0)

    g2sum = jnp.reshape(t, (_G2, 8, _HIDDEN))[:, 7, :]
    g2last = jnp.reshape(g1last, (_G2, 8, 1))[:, 7, :]
    u = g2sum
    for d in (1, 2, 4, 8, 16, 32):
        u = _seg_scan_step(u, g2last, d)
    l3prev = jnp.concatenate(
        [jnp.zeros((1, _HIDDEN), jnp.float32), u[:-1, :]], axis=0)

    def _window(w, carry):
        base = w * _S
        basef = base.astype(jnp.float32)
        iota_r = jax.lax.broadcasted_iota(
            jnp.int32, (_S, _R), 0).astype(jnp.float32)
        oh = (brow - basef == iota_r).astype(jnp.float32)
        sums_u = jax.lax.dot_general(oh, attended, (((1,), (0,)), ((), ())),
                                     preferred_element_type=jnp.float32)
        cnts_u = jnp.sum(oh, axis=1, keepdims=True)
        sel1 = oh * rend
        max1 = jax.lax.dot_general(sel1, scanned1, (((1,), (0,)), ((), ())),
                                   preferred_element_type=jnp.float32)
        iota_g1 = jax.lax.broadcasted_iota(
            jnp.int32, (_S, _G1), 0).astype(jnp.float32)
        ohg1 = (g1first - basef == iota_g1).astype(jnp.float32) * cond1
        max2 = jax.lax.dot_general(ohg1, l2prev, (((1,), (0,)), ((), ())),
                                   preferred_element_type=jnp.float32)
        iota_g2 = jax.lax.broadcasted_iota(
            jnp.int32, (_S, _G2), 0).astype(jnp.float32)
        ohg2 = (g2first - basef == iota_g2).astype(jnp.float32) * cond2
        max3 = jax.lax.dot_general(ohg2, l3prev, (((1,), (0,)), ((), ())),
                                   preferred_element_type=jnp.float32)
        maxs_u = jnp.maximum(jnp.maximum(max1, max2), max3)
        sum_s[pl.ds(base, _S), :] += sums_u
        cnt_s[pl.ds(base, _S), :] += cnts_u
        max_s[pl.ds(base, _S), :] = jnp.maximum(max_s[pl.ds(base, _S), :],
                                                maxs_u)
        return carry

    jax.lax.fori_loop(wlo_ref[i], whi_ref[i] + 1, _window, 0)

    @pl.when(i == nb - 1)
    def _finish():
        cnt = cnt_s[...]
        out_ref[:, :_HIDDEN] = max_s[...]
        out_ref[:, _HIDDEN:] = sum_s[...] / jnp.maximum(cnt, 1.0)


def _shift_fill(a, d, fill):
    return jnp.concatenate([jnp.full((d,), fill, a.dtype), a[:-d]])


def _carry_cond(gfirst, glast, gidx, glen):
    gprev = jnp.where(gidx >= 1, _shift_fill(glast, 1, -1), -1)
    gnext = jnp.concatenate([gfirst[1:], jnp.full((1,), -1, gfirst.dtype)])
    return ((gprev == gfirst)
            & ((gidx == glen - 1) | (gnext != gfirst))).astype(jnp.float32)


@jax.jit
def kernel(x, batch, W1, b1, W2, b2):
    n = x.shape[0]
    assert n % _R == 0
    nb = n // _R
    batch = batch.astype(jnp.int32)
    bcol = batch.reshape(nb, _R, 1)
    wlo = (batch[::_R] // _S).astype(jnp.int32)
    whi = (batch[_R - 1::_R] // _S).astype(jnp.int32)

    ridx = jnp.arange(n, dtype=jnp.int32) % _R
    nxt = jnp.concatenate([batch[1:], jnp.full((1,), -1, jnp.int32)])
    rend = ((batch != nxt) | (ridx == _R - 1)).astype(jnp.float32)

    b8 = batch.reshape(-1, 8)
    g1first, g1last = b8[:, 0], b8[:, 7]
    g1idx = jnp.arange(n // 8, dtype=jnp.int32) % _G1
    cond1 = _carry_cond(g1first, g1last, g1idx, _G1)

    g2first = g1first.reshape(-1, 8)[:, 0]
    g2last = g1last.reshape(-1, 8)[:, 7]
    g2idx = jnp.arange(n // 64, dtype=jnp.int32) % _G2
    cond2 = _carry_cond(g2first, g2last, g2idx, _G2)

    pad1 = jnp.full((nb, 512 - _G1), -1.0, jnp.float32)
    pad2 = jnp.full((nb, 128 - _G2), -1.0, jnp.float32)
    rowpack = jnp.concatenate([
        batch.astype(jnp.float32).reshape(nb, _R),
        rend.reshape(nb, _R),
        g1first.astype(jnp.float32).reshape(nb, _G1), pad1,
        cond1.reshape(nb, _G1), jnp.zeros((nb, 512 - _G1), jnp.float32),
        g2first.astype(jnp.float32).reshape(nb, _G2), pad2,
        cond2.reshape(nb, _G2), jnp.zeros((nb, 128 - _G2), jnp.float32),
    ], axis=1).reshape(nb, 1, _RP)

    b1r = b1.reshape(1, _HIDDEN)
    b2r = b2.reshape(1, _HIDDEN)

    grid_spec = pltpu.PrefetchScalarGridSpec(
        num_scalar_prefetch=2,
        grid=(nb,),
        in_specs=[
            pl.BlockSpec((_R, _HIDDEN), lambda i, *_: (i, 0)),
            pl.BlockSpec((1, _R, 1), lambda i, *_: (i, 0, 0)),
            pl.BlockSpec((1, 1, _RP), lambda i, *_: (i, 0, 0)),
            pl.BlockSpec((_HIDDEN, _HIDDEN), lambda i, *_: (0, 0)),
            pl.BlockSpec((1, _HIDDEN), lambda i, *_: (0, 0)),
            pl.BlockSpec((_HIDDEN, _HIDDEN), lambda i, *_: (0, 0)),
            pl.BlockSpec((1, _HIDDEN), lambda i, *_: (0, 0)),
        ],
        out_specs=pl.BlockSpec((_NSEG, 2 * _HIDDEN), lambda i, *_: (0, 0)),
        scratch_shapes=[
            pltpu.VMEM((_NSEG, _HIDDEN), jnp.float32),
            pltpu.VMEM((_NSEG, _HIDDEN), jnp.float32),
            pltpu.VMEM((_NSEG, 1), jnp.float32),
        ],
    )
    out = pl.pallas_call(
        _fused_kernel,
        grid_spec=grid_spec,
        out_shape=jax.ShapeDtypeStruct((_NSEG, 2 * _HIDDEN), jnp.float32),
        compiler_params=pltpu.CompilerParams(
            dimension_semantics=("arbitrary",)),
    )(wlo, whi, x, bcol, rowpack, W1, b1r, W2, b2r)
    return out


# speedup vs baseline: 1.0345x; 1.0345x over previous
import jax
import jax.numpy as jnp
from jax.experimental import pallas as pl
from jax.experimental.pallas import tpu as pltpu

_HIDDEN = 128
_NSEG = 1024
_R = 3200
_G = _R // 8
_S = 64
_NL2 = 9
_CM = 3 * _R + _NL2 * _G
_RP = 2 * _R + 2 * _G


def _fused_kernel(wlo_ref, whi_ref, x_ref, cm_ref, rp_ref, w1_ref, b1_ref,
                  w2_ref, b2_ref, out_ref, sum_s, max_s, cnt_s):
    i = pl.program_id(0)
    nb = pl.num_programs(0)

    @pl.when(i == 0)
    def _init():
        sum_s[...] = jnp.zeros_like(sum_s)
        max_s[...] = jnp.zeros_like(max_s)
        cnt_s[...] = jnp.zeros_like(cnt_s)

    x = x_ref[...]
    h = jax.lax.dot_general(x, w1_ref[...], (((1,), (1,)), ((), ())),
                            preferred_element_type=jnp.float32)
    h = jnp.maximum(h + b1_ref[...], 0.0)
    att = jax.lax.dot_general(h, w2_ref[...], (((1,), (1,)), ((), ())),
                              preferred_element_type=jnp.float32)
    att = jax.nn.sigmoid(att + b2_ref[...])
    attended = h * att

    cm = cm_ref[0]
    rp = rp_ref[0]
    brow = rp[:, 0:_R]
    rend = rp[:, _R:2 * _R]
    gfirst = rp[:, 2 * _R:2 * _R + _G]
    cond = rp[:, 2 * _R + _G:2 * _R + 2 * _G]

    s = attended.astype(jnp.bfloat16)
    for k, d in enumerate((1, 2, 4)):
        sh = jnp.concatenate(
            [jnp.zeros((d, _HIDDEN), jnp.bfloat16), s[:-d, :]], axis=0)
        s = jnp.maximum(s, sh * cm[k * _R:(k + 1) * _R, :])
    scanned1 = s

    gsum = jnp.reshape(scanned1, (_G, 8, _HIDDEN))[:, 7, :]

    t = gsum
    d = 1
    for k in range(_NL2):
        sh = jnp.concatenate(
            [jnp.zeros((d, _HIDDEN), jnp.bfloat16), t[:-d, :]], axis=0)
        base = 3 * _R + k * _G
        t = jnp.maximum(t, sh * cm[base:base + _G, :])
        d *= 2
    gscan_prev = jnp.concatenate(
        [jnp.zeros((1, _HIDDEN), jnp.bfloat16), t[:-1, :]], axis=0)

    def _window(w, carry):
        base = w * _S
        basef = base.astype(jnp.float32)
        iota_r = jax.lax.broadcasted_iota(
            jnp.int32, (_S, _R), 0).astype(jnp.float32)
        oh = (brow - basef == iota_r).astype(jnp.float32)
        sums_u = jax.lax.dot_general(oh, attended, (((1,), (0,)), ((), ())),
                                     preferred_element_type=jnp.float32)
        cnts_u = jnp.sum(oh, axis=1, keepdims=True)
        sel1 = (oh * rend).astype(jnp.bfloat16)
        max1 = jax.lax.dot_general(sel1, scanned1, (((1,), (0,)), ((), ())),
                                   preferred_element_type=jnp.float32)
        iota_g = jax.lax.broadcasted_iota(
            jnp.int32, (_S, _G), 0).astype(jnp.float32)
        ohg = ((gfirst - basef == iota_g).astype(jnp.float32)
               * cond).astype(jnp.bfloat16)
        max2 = jax.lax.dot_general(ohg, gscan_prev, (((1,), (0,)), ((), ())),
                                   preferred_element_type=jnp.float32)
        maxs_u = jnp.maximum(max1, max2)
        sum_s[pl.ds(base, _S), :] += sums_u
        cnt_s[pl.ds(base, _S), :] += cnts_u
        max_s[pl.ds(base, _S), :] = jnp.maximum(max_s[pl.ds(base, _S), :],
                                                maxs_u)
        return carry

    jax.lax.fori_loop(wlo_ref[i], whi_ref[i] + 1, _window, 0)

    @pl.when(i == nb - 1)
    def _finish():
        cnt = cnt_s[...]
        out_ref[:, :_HIDDEN] = max_s[...]
        out_ref[:, _HIDDEN:] = sum_s[...] / jnp.maximum(cnt, 1.0)


def _shift_fill(a, d, fill):
    return jnp.concatenate([jnp.full((d,), fill, a.dtype), a[:-d]])


@jax.jit
def kernel(x, batch, W1, b1, W2, b2):
    n = x.shape[0]
    assert n % _R == 0
    nb = n // _R
    batch = batch.astype(jnp.int32)
    wlo = (batch[::_R] // _S).astype(jnp.int32)
    whi = (batch[_R - 1::_R] // _S).astype(jnp.int32)

    ridx = jnp.arange(n, dtype=jnp.int32) % _R
    bblk = batch.reshape(nb, _R)
    mcols = [((ridx >= d) & (batch == _shift_fill(batch, d, -1)))
             .astype(jnp.float32).reshape(nb, _R) for d in (1, 2, 4)]

    nxt = jnp.concatenate([batch[1:], jnp.full((1,), -1, jnp.int32)])
    rend = ((batch != nxt) | (ridx == _R - 1)).astype(jnp.float32)

    glast = batch[7::8]
    gfirst = batch[0::8]
    gidx = jnp.arange(n // 8, dtype=jnp.int32) % _G
    l2cols = [((gidx >= d) & (glast == _shift_fill(glast, d, -1)))
              .astype(jnp.float32).reshape(nb, _G)
              for d in (1, 2, 4, 8, 16, 32, 64, 128, 256)]

    glast_prev = jnp.where(gidx >= 1, _shift_fill(glast, 1, -1), -1)
    gfirst_next = jnp.concatenate([gfirst[1:], jnp.full((1,), -1, jnp.int32)])
    cond = ((glast_prev == gfirst)
            & ((gidx == _G - 1) | (gfirst_next != gfirst))).astype(jnp.float32)

    colpack = (jnp.concatenate(mcols + l2cols, axis=1)
               .astype(jnp.bfloat16).reshape(nb, _CM, 1))
    rowpack = jnp.concatenate(
        [bblk.astype(jnp.float32), rend.reshape(nb, _R),
         gfirst.astype(jnp.float32).reshape(nb, _G), cond.reshape(nb, _G)],
        axis=1).reshape(nb, 1, _RP)

    b1r = b1.reshape(1, _HIDDEN)
    b2r = b2.reshape(1, _HIDDEN)

    grid_spec = pltpu.PrefetchScalarGridSpec(
        num_scalar_prefetch=2,
        grid=(nb,),
        in_specs=[
            pl.BlockSpec((_R, _HIDDEN), lambda i, *_: (i, 0)),
            pl.BlockSpec((1, _CM, 1), lambda i, *_: (i, 0, 0)),
            pl.BlockSpec((1, 1, _RP), lambda i, *_: (i, 0, 0)),
            pl.BlockSpec((_HIDDEN, _HIDDEN), lambda i, *_: (0, 0)),
            pl.BlockSpec((1, _HIDDEN), lambda i, *_: (0, 0)),
            pl.BlockSpec((_HIDDEN, _HIDDEN), lambda i, *_: (0, 0)),
            pl.BlockSpec((1, _HIDDEN), lambda i, *_: (0, 0)),
        ],
        out_specs=pl.BlockSpec((_NSEG, 2 * _HIDDEN), lambda i, *_: (0, 0)),
        scratch_shapes=[
            pltpu.VMEM((_NSEG, _HIDDEN), jnp.float32),
            pltpu.VMEM((_NSEG, _HIDDEN), jnp.float32),
            pltpu.VMEM((_NSEG, 1), jnp.float32),
        ],
    )
    out = pl.pallas_call(
        _fused_kernel,
        grid_spec=grid_spec,
        out_shape=jax.ShapeDtypeStruct((_NSEG, 2 * _HIDDEN), jnp.float32),
        compiler_params=pltpu.CompilerParams(
            dimension_semantics=("arbitrary",)),
    )(wlo, whi, x, colpack, rowpack, W1, b1r, W2, b2r)
    return out


# speedup vs baseline: 1.3788x; 1.3328x over previous
import jax
import jax.numpy as jnp
from jax.experimental import pallas as pl
from jax.experimental.pallas import tpu as pltpu

_HIDDEN = 128
_NSEG = 1024
_R = 3200
_G = _R // 8
_S = 64
_NL2 = 9
_GP = 512
_OFF_REND = _R
_OFF_M = 2 * _R
_OFF_L2 = 5 * _R
_OFF_GF = _OFF_L2 + _NL2 * _GP
_OFF_C1 = _OFF_GF + _GP
_RP = _OFF_C1 + _GP


def _fused_kernel(wlo_ref, whi_ref, x_ref, rp_ref, e_ref, w1_ref, b1_ref,
                  w2_ref, b2_ref, out_ref, sum_s, max_s, cnt_s):
    i = pl.program_id(0)
    nb = pl.num_programs(0)

    @pl.when(i == 0)
    def _init():
        sum_s[...] = jnp.zeros_like(sum_s)
        max_s[...] = jnp.zeros_like(max_s)
        cnt_s[...] = jnp.zeros_like(cnt_s)

    x = x_ref[...]
    h = jax.lax.dot_general(x, w1_ref[...], (((1,), (1,)), ((), ())),
                            preferred_element_type=jnp.float32)
    h = jnp.maximum(h + b1_ref[...], 0.0)
    att = jax.lax.dot_general(h, w2_ref[...], (((1,), (1,)), ((), ())),
                              preferred_element_type=jnp.float32)
    att = jax.nn.sigmoid(att + b2_ref[...])
    attended = h * att

    rp = rp_ref[0]
    brow = rp[:, 0:_R]
    rend = rp[:, _OFF_REND:_OFF_REND + _R]
    gfirst = rp[:, _OFF_GF:_OFF_GF + _G]
    cond = rp[:, _OFF_C1:_OFF_C1 + _G]

    eye = (jax.lax.broadcasted_iota(jnp.int32, (_HIDDEN, _HIDDEN), 0)
           == jax.lax.broadcasted_iota(jnp.int32, (_HIDDEN, _HIDDEN), 1)
           ).astype(jnp.float32)
    attT = jax.lax.dot_general(eye, attended, (((1,), (1,)), ((), ())),
                               preferred_element_type=jnp.float32)

    s = attT
    for k, d in enumerate((1, 2, 4)):
        sh = jnp.concatenate(
            [jnp.zeros((_HIDDEN, d), jnp.float32), s[:, :-d]], axis=1)
        off = _OFF_M + k * _R
        s = jnp.maximum(s, sh * rp[:, off:off + _R])
    scannedT = s

    gsumT = jax.lax.dot_general(scannedT, e_ref[...], (((1,), (0,)), ((), ())),
                                preferred_element_type=jnp.float32)

    t = gsumT
    d = 1
    for k in range(_NL2):
        sh = jnp.concatenate(
            [jnp.zeros((_HIDDEN, d), jnp.float32), t[:, :-d]], axis=1)
        off = _OFF_L2 + k * _GP
        t = jnp.maximum(t, sh * rp[:, off:off + _G])
        d *= 2
    gprevT = jnp.concatenate(
        [jnp.zeros((_HIDDEN, 1), jnp.float32), t[:, :-1]], axis=1)

    def _window(w, carry):
        base = w * _S
        basef = base.astype(jnp.float32)
        iota_r = jax.lax.broadcasted_iota(
            jnp.int32, (_S, _R), 0).astype(jnp.float32)
        oh = (brow - basef == iota_r).astype(jnp.float32)
        sums_u = jax.lax.dot_general(oh, attended, (((1,), (0,)), ((), ())),
                                     preferred_element_type=jnp.float32)
        cnts_u = jnp.sum(oh, axis=1, keepdims=True)
        sel1 = oh * rend
        max1 = jax.lax.dot_general(sel1, scannedT, (((1,), (1,)), ((), ())),
                                   preferred_element_type=jnp.float32)
        iota_g = jax.lax.broadcasted_iota(
            jnp.int32, (_S, _G), 0).astype(jnp.float32)
        ohg = (gfirst - basef == iota_g).astype(jnp.float32) * cond
        max2 = jax.lax.dot_general(ohg, gprevT, (((1,), (1,)), ((), ())),
                                   preferred_element_type=jnp.float32)
        maxs_u = jnp.maximum(max1, max2)
        sum_s[pl.ds(base, _S), :] += sums_u
        cnt_s[pl.ds(base, _S), :] += cnts_u
        max_s[pl.ds(base, _S), :] = jnp.maximum(max_s[pl.ds(base, _S), :],
                                                maxs_u)
        return carry

    jax.lax.fori_loop(wlo_ref[i], whi_ref[i] + 1, _window, 0)

    @pl.when(i == nb - 1)
    def _finish():
        cnt = cnt_s[...]
        out_ref[:, :_HIDDEN] = max_s[...]
        out_ref[:, _HIDDEN:] = sum_s[...] / jnp.maximum(cnt, 1.0)


def _shift_fill(a, d, fill):
    return jnp.concatenate([jnp.full((d,), fill, a.dtype), a[:-d]])


@jax.jit
def kernel(x, batch, W1, b1, W2, b2):
    n = x.shape[0]
    assert n % _R == 0
    nb = n // _R
    batch = batch.astype(jnp.int32)
    wlo = (batch[::_R] // _S).astype(jnp.int32)
    whi = (batch[_R - 1::_R] // _S).astype(jnp.int32)

    ridx = jnp.arange(n, dtype=jnp.int32) % _R
    mcols = [((ridx >= d) & (batch == _shift_fill(batch, d, -1)))
             .astype(jnp.float32).reshape(nb, _R) for d in (1, 2, 4)]

    nxt = jnp.concatenate([batch[1:], jnp.full((1,), -1, jnp.int32)])
    rend = ((batch != nxt) | (ridx == _R - 1)).astype(jnp.float32)

    b8 = batch.reshape(-1, 8)
    gfirst, glast = b8[:, 0], b8[:, 7]
    gidx = jnp.arange(n // 8, dtype=jnp.int32) % _G
    l2cols = [((gidx >= d) & (glast == _shift_fill(glast, d, -1)))
              .astype(jnp.float32).reshape(nb, _G)
              for d in (1, 2, 4, 8, 16, 32, 64, 128, 256)]

    gprev = jnp.where(gidx >= 1, _shift_fill(glast, 1, -1), -1)
    gnext = jnp.concatenate([gfirst[1:], jnp.full((1,), -1, jnp.int32)])
    cond = ((gprev == gfirst)
            & ((gidx == _G - 1) | (gnext != gfirst))).astype(jnp.float32)

    zpad = jnp.zeros((nb, _GP - _G), jnp.float32)
    pieces = [batch.astype(jnp.float32).reshape(nb, _R),
              rend.reshape(nb, _R)] + mcols
    for c in l2cols:
        pieces += [c, zpad]
    pieces += [gfirst.astype(jnp.float32).reshape(nb, _G),
               jnp.full((nb, _GP - _G), -1.0, jnp.float32),
               cond.reshape(nb, _G), zpad]
    rowpack = jnp.concatenate(pieces, axis=1).reshape(nb, 1, _RP)

    emat = (jnp.arange(_R, dtype=jnp.int32)[:, None]
            == (jnp.arange(_G, dtype=jnp.int32) * 8 + 7)[None, :]
            ).astype(jnp.float32)

    b1r = b1.reshape(1, _HIDDEN)
    b2r = b2.reshape(1, _HIDDEN)

    grid_spec = pltpu.PrefetchScalarGridSpec(
        num_scalar_prefetch=2,
        grid=(nb,),
        in_specs=[
            pl.BlockSpec((_R, _HIDDEN), lambda i, *_: (i, 0)),
            pl.BlockSpec((1, 1, _RP), lambda i, *_: (i, 0, 0)),
            pl.BlockSpec((_R, _G), lambda i, *_: (0, 0)),
            pl.BlockSpec((_HIDDEN, _HIDDEN), lambda i, *_: (0, 0)),
            pl.BlockSpec((1, _HIDDEN), lambda i, *_: (0, 0)),
            pl.BlockSpec((_HIDDEN, _HIDDEN), lambda i, *_: (0, 0)),
            pl.BlockSpec((1, _HIDDEN), lambda i, *_: (0, 0)),
        ],
        out_specs=pl.BlockSpec((_NSEG, 2 * _HIDDEN), lambda i, *_: (0, 0)),
        scratch_shapes=[
            pltpu.VMEM((_NSEG, _HIDDEN), jnp.float32),
            pltpu.VMEM((_NSEG, _HIDDEN), jnp.float32),
            pltpu.VMEM((_NSEG, 1), jnp.float32),
        ],
    )
    out = pl.pallas_call(
        _fused_kernel,
        grid_spec=grid_spec,
        out_shape=jax.ShapeDtypeStruct((_NSEG, 2 * _HIDDEN), jnp.float32),
        compiler_params=pltpu.CompilerParams(
            dimension_semantics=("arbitrary",)),
    )(wlo, whi, x, rowpack, emat, W1, b1r, W2, b2r)
    return out


# speedup vs baseline: 1.7231x; 1.2497x over previous
import jax
import jax.numpy as jnp
from jax.experimental import pallas as pl
from jax.experimental.pallas import tpu as pltpu

_HIDDEN = 128
_NSEG = 1024
_R = 3200
_G = _R // 8
_S = 64
_NL2 = 9


def _shift_lanes(v, d, fill):
    return jnp.concatenate(
        [jnp.full((1, d), fill, v.dtype), v[:, :-d]], axis=1)


def _fused_kernel(wlo_ref, whi_ref, x_ref, brow_ref, e7_ref, e0_ref, w1_ref,
                  b1_ref, w2_ref, b2_ref, out_ref, sum_s, max_s, cnt_s):
    i = pl.program_id(0)
    nb = pl.num_programs(0)

    @pl.when(i == 0)
    def _init():
        sum_s[...] = jnp.zeros_like(sum_s)
        max_s[...] = jnp.zeros_like(max_s)
        cnt_s[...] = jnp.zeros_like(cnt_s)

    x = x_ref[...]
    h = jax.lax.dot_general(x, w1_ref[...], (((1,), (1,)), ((), ())),
                            preferred_element_type=jnp.float32)
    h = jnp.maximum(h + b1_ref[...], 0.0)
    att = jax.lax.dot_general(h, w2_ref[...], (((1,), (1,)), ((), ())),
                              preferred_element_type=jnp.float32)
    att = jax.nn.sigmoid(att + b2_ref[...])
    attended = h * att

    brow = brow_ref[0]
    lane_r = jax.lax.broadcasted_iota(jnp.int32, (1, _R), 1)

    nxt = jnp.concatenate(
        [brow[:, 1:], jnp.full((1, 1), -1.0, jnp.float32)], axis=1)
    rend = ((brow != nxt) | (lane_r == _R - 1)).astype(jnp.float32)

    glast = jax.lax.dot_general(brow, e7_ref[...], (((1,), (0,)), ((), ())),
                                preferred_element_type=jnp.float32)
    gfirst = jax.lax.dot_general(brow, e0_ref[...], (((1,), (0,)), ((), ())),
                                 preferred_element_type=jnp.float32)
    lane_g = jax.lax.broadcasted_iota(jnp.int32, (1, _G), 1)
    gprev_id = _shift_lanes(glast, 1, -1.0)
    gnext = jnp.concatenate(
        [gfirst[:, 1:], jnp.full((1, 1), -1.0, jnp.float32)], axis=1)
    cond = ((gprev_id == gfirst)
            & ((lane_g == _G - 1) | (gnext != gfirst))).astype(jnp.float32)

    eye = (jax.lax.broadcasted_iota(jnp.int32, (_HIDDEN, _HIDDEN), 0)
           == jax.lax.broadcasted_iota(jnp.int32, (_HIDDEN, _HIDDEN), 1)
           ).astype(jnp.float32)
    attT = jax.lax.dot_general(eye, attended, (((1,), (1,)), ((), ())),
                               preferred_element_type=jnp.float32)

    s = attT
    for d in (1, 2, 4):
        m = ((brow == _shift_lanes(brow, d, -1.0))
             & (lane_r >= d)).astype(jnp.float32)
        sh = jnp.concatenate(
            [jnp.zeros((_HIDDEN, d), jnp.float32), s[:, :-d]], axis=1)
        s = jnp.maximum(s, sh * m)
    scannedT = s

    gsumT = jax.lax.dot_general(scannedT, e7_ref[...], (((1,), (0,)), ((), ())),
                                preferred_element_type=jnp.float32)

    t = gsumT
    d = 1
    for _ in range(_NL2):
        m = ((glast == _shift_lanes(glast, d, -1.0))
             & (lane_g >= d)).astype(jnp.float32)
        sh = jnp.concatenate(
            [jnp.zeros((_HIDDEN, d), jnp.float32), t[:, :-d]], axis=1)
        t = jnp.maximum(t, sh * m)
        d *= 2
    gprevT = jnp.concatenate(
        [jnp.zeros((_HIDDEN, 1), jnp.float32), t[:, :-1]], axis=1)

    def _window(w, carry):
        base = w * _S
        basef = base.astype(jnp.float32)
        iota_r = jax.lax.broadcasted_iota(
            jnp.int32, (_S, _R), 0).astype(jnp.float32)
        oh = (brow - basef == iota_r).astype(jnp.float32)
        sums_u = jax.lax.dot_general(oh, attended, (((1,), (0,)), ((), ())),
                                     preferred_element_type=jnp.float32)
        cnts_u = jnp.sum(oh, axis=1, keepdims=True)
        sel1 = oh * rend
        max1 = jax.lax.dot_general(sel1, scannedT, (((1,), (1,)), ((), ())),
                                   preferred_element_type=jnp.float32)
        iota_g = jax.lax.broadcasted_iota(
            jnp.int32, (_S, _G), 0).astype(jnp.float32)
        ohg = (gfirst - basef == iota_g).astype(jnp.float32) * cond
        max2 = jax.lax.dot_general(ohg, gprevT, (((1,), (1,)), ((), ())),
                                   preferred_element_type=jnp.float32)
        maxs_u = jnp.maximum(max1, max2)
        sum_s[pl.ds(base, _S), :] += sums_u
        cnt_s[pl.ds(base, _S), :] += cnts_u
        max_s[pl.ds(base, _S), :] = jnp.maximum(max_s[pl.ds(base, _S), :],
                                                maxs_u)
        return carry

    jax.lax.fori_loop(wlo_ref[i], whi_ref[i] + 1, _window, 0)

    @pl.when(i == nb - 1)
    def _finish():
        cnt = cnt_s[...]
        out_ref[:, :_HIDDEN] = max_s[...]
        out_ref[:, _HIDDEN:] = sum_s[...] / jnp.maximum(cnt, 1.0)


@jax.jit
def kernel(x, batch, W1, b1, W2, b2):
    n = x.shape[0]
    assert n % _R == 0
    nb = n // _R
    batch = batch.astype(jnp.int32)
    wlo = (batch[::_R] // _S).astype(jnp.int32)
    whi = (batch[_R - 1::_R] // _S).astype(jnp.int32)
    brow = batch.astype(jnp.float32).reshape(nb, 1, _R)

    r_ids = jnp.arange(_R, dtype=jnp.int32)[:, None]
    g_ids = jnp.arange(_G, dtype=jnp.int32)[None, :]
    e7 = (r_ids == g_ids * 8 + 7).astype(jnp.float32)
    e0 = (r_ids == g_ids * 8).astype(jnp.float32)

    b1r = b1.reshape(1, _HIDDEN)
    b2r = b2.reshape(1, _HIDDEN)

    grid_spec = pltpu.PrefetchScalarGridSpec(
        num_scalar_prefetch=2,
        grid=(nb,),
        in_specs=[
            pl.BlockSpec((_R, _HIDDEN), lambda i, *_: (i, 0)),
            pl.BlockSpec((1, 1, _R), lambda i, *_: (i, 0, 0)),
            pl.BlockSpec((_R, _G), lambda i, *_: (0, 0)),
            pl.BlockSpec((_R, _G), lambda i, *_: (0, 0)),
            pl.BlockSpec((_HIDDEN, _HIDDEN), lambda i, *_: (0, 0)),
            pl.BlockSpec((1, _HIDDEN), lambda i, *_: (0, 0)),
            pl.BlockSpec((_HIDDEN, _HIDDEN), lambda i, *_: (0, 0)),
            pl.BlockSpec((1, _HIDDEN), lambda i, *_: (0, 0)),
        ],
        out_specs=pl.BlockSpec((_NSEG, 2 * _HIDDEN), lambda i, *_: (0, 0)),
        scratch_shapes=[
            pltpu.VMEM((_NSEG, _HIDDEN), jnp.float32),
            pltpu.VMEM((_NSEG, _HIDDEN), jnp.float32),
            pltpu.VMEM((_NSEG, 1), jnp.float32),
        ],
    )
    out = pl.pallas_call(
        _fused_kernel,
        grid_spec=grid_spec,
        out_shape=jax.ShapeDtypeStruct((_NSEG, 2 * _HIDDEN), jnp.float32),
        compiler_params=pltpu.CompilerParams(
            dimension_semantics=("arbitrary",)),
    )(wlo, whi, x, brow, e7, e0, W1, b1r, W2, b2r)
    return out
